# baseline (speedup 1.0000x reference)
import functools

import jax
import jax.numpy as jnp
from jax import lax
from jax.experimental import pallas as pl
from jax.experimental.pallas import tpu as pltpu
from jax.experimental.pallas import tpu_sc as plsc

N = 10000
M = 5000
NNZ = 320000
DIN = 128
DOUT = 128
DH = DOUT // 2
NC = 2
NS = 16
MP = 5120
NP = 10240
PT = NNZ // NS
B = 400
NK = PT // B
RC = 64
ET = MP // NS
VT = NP // NS


def _mm_body(x_ref, w_ref, b_ref, o_ref):
    o_ref[...] = lax.dot_general(
        x_ref[...], w_ref[...], (((1,), (1,)), ((), ())),
        preferred_element_type=jnp.float32) + b_ref[0]


def _project(X, W, b):
    blk = 1000
    nblk = N // blk
    return pl.pallas_call(
        _mm_body,
        grid=(NC, nblk),
        in_specs=[
            pl.BlockSpec((blk, DIN), lambda c, i: (i, 0)),
            pl.BlockSpec((DH, DIN), lambda c, i: (c, 0)),
            pl.BlockSpec((1, 1, DH), lambda c, i: (c, 0, 0)),
        ],
        out_specs=pl.BlockSpec((blk, DH), lambda c, i: (c * nblk + i, 0)),
        out_shape=jax.ShapeDtypeStruct((NC * N, DH), jnp.float32),
    )(X, W, b.reshape(NC, 1, DH))


def _sc_body(h_ref, vi_ref, ei_ref, out_ref,
             e_acc, v_acc, e_cnt, v_cnt,
             vidx0, vadj0, eidx0, rows0, vidx1, vadj1, eidx1, rows1,
             ones, zbuf, dbuf, cbuf, z1,
             sem_g, sem_s0):
    cid = lax.axis_index("c")
    sid = lax.axis_index("s")

    one16 = jnp.full((16,), 1.0, jnp.float32)
    zero16 = jnp.zeros((16,), jnp.float32)

    def init_ones(i, c):
        ones[pl.ds(i * 16, 16)] = one16
        return c
    lax.fori_loop(0, B // 16, init_ones, 0)

    def init_z(i, c):
        zbuf[i // 4, pl.ds((i % 4) * 16, 16)] = zero16
        return c
    lax.fori_loop(0, RC * 4, init_z, 0)

    def init_z1(i, c):
        z1[pl.ds(i * 16, 16)] = zero16
        return c
    lax.fori_loop(0, VT // 16, init_z1, 0)

    def zero_e(j, c):
        pltpu.sync_copy(zbuf, e_acc.at[pl.ds(sid * ET + j * RC, RC)])
        return c
    lax.fori_loop(0, ET // RC, zero_e, 0)

    def zero_v(j, c):
        pltpu.sync_copy(zbuf, v_acc.at[pl.ds(sid * VT + j * RC, RC)])
        return c
    lax.fori_loop(0, VT // RC, zero_v, 0)

    pltpu.sync_copy(z1.at[pl.ds(0, ET)], e_cnt.at[pl.ds(sid * ET, ET)])
    pltpu.sync_copy(z1, v_cnt.at[pl.ds(sid * VT, VT)])
    plsc.subcore_barrier()

    pair0 = sid * PT
    off = cid * N

    def load_adj(k, vidx, vadj, eidx):
        base = pair0 + k * B
        pltpu.sync_copy(vi_ref.at[pl.ds(base, B)], vidx)
        pltpu.sync_copy(ei_ref.at[pl.ds(base, B)], eidx)

        def adj(i, cc):
            vadj[pl.ds(i * 16, 16)] = vidx[pl.ds(i * 16, 16)] + off
            return cc
        lax.fori_loop(0, B // 16, adj, 0)

    def chunk_a(k, vidx, vadj, eidx, rows, sem_gc,
                nvidx, nvadj, neidx, nrows, nsem_g):
        @pl.when(k + 1 < NK)
        def _():
            load_adj(k + 1, nvidx, nvadj, neidx)
            pltpu.async_copy(h_ref.at[nvadj], nrows, nsem_g)
        pltpu.sync_copy(ones, v_cnt.at[vidx], add=True)
        pltpu.sync_copy(ones, e_cnt.at[eidx], add=True)
        pltpu.make_async_copy(h_ref.at[vadj], rows, sem_gc).wait()
        pltpu.sync_copy(rows, e_acc.at[eidx], add=True)

    load_adj(0, vidx0, vadj0, eidx0)
    pltpu.async_copy(h_ref.at[vadj0], rows0, sem_g)

    def pair_a(i, c):
        chunk_a(2 * i, vidx0, vadj0, eidx0, rows0, sem_g,
                vidx1, vadj1, eidx1, rows1, sem_s0)
        chunk_a(2 * i + 1, vidx1, vadj1, eidx1, rows1, sem_s0,
                vidx0, vadj0, eidx0, rows0, sem_g)
        return c
    lax.fori_loop(0, NK // 2, pair_a, 0)
    plsc.subcore_barrier()

    def div_e(j, c):
        r0 = sid * ET + j * RC
        pltpu.sync_copy(e_acc.at[pl.ds(r0, RC)], dbuf)
        pltpu.sync_copy(e_cnt.at[pl.ds(r0, RC)], cbuf)

        def grp(g, cc):
            recv = 1.0 / jnp.maximum(cbuf[pl.ds(g * 16, 16)], 1.0)
            for r2 in range(16):
                r = g * 16 + r2
                rec = recv[r2]
                for q in range(DH // 16):
                    dbuf[r, pl.ds(q * 16, 16)] = dbuf[r, pl.ds(q * 16, 16)] * rec
            return cc
        lax.fori_loop(0, RC // 16, grp, 0)
        pltpu.sync_copy(dbuf, e_acc.at[pl.ds(r0, RC)])
        return c
    lax.fori_loop(0, ET // RC, div_e, 0)
    plsc.subcore_barrier()

    def load_b(k, vidx, eidx):
        base = pair0 + k * B
        pltpu.sync_copy(vi_ref.at[pl.ds(base, B)], vidx)
        pltpu.sync_copy(ei_ref.at[pl.ds(base, B)], eidx)

    def chunk_b(k, vidx, eidx, rows, sem_gc, nvidx, neidx, nrows, nsem_g):
        @pl.when(k + 1 < NK)
        def _():
            load_b(k + 1, nvidx, neidx)
            pltpu.async_copy(e_acc.at[neidx], nrows, nsem_g)
        pltpu.make_async_copy(e_acc.at[eidx], rows, sem_gc).wait()
        pltpu.sync_copy(rows, v_acc.at[vidx], add=True)

    load_b(0, vidx0, eidx0)
    pltpu.async_copy(e_acc.at[eidx0], rows0, sem_g)

    def pair_b(i, c):
        chunk_b(2 * i, vidx0, eidx0, rows0, sem_g,
                vidx1, eidx1, rows1, sem_s0)
        chunk_b(2 * i + 1, vidx1, eidx1, rows1, sem_s0,
                vidx0, eidx0, rows0, sem_g)
        return c
    lax.fori_loop(0, NK // 2, pair_b, 0)
    plsc.subcore_barrier()

    def div_v(j, c):
        r0 = sid * VT + j * RC
        pltpu.sync_copy(v_acc.at[pl.ds(r0, RC)], dbuf)
        pltpu.sync_copy(v_cnt.at[pl.ds(r0, RC)], cbuf)

        def grp(g, cc):
            recv = 1.0 / jnp.maximum(cbuf[pl.ds(g * 16, 16)], 1.0)
            for r2 in range(16):
                r = g * 16 + r2
                rec = recv[r2]
                for q in range(DH // 16):
                    v = dbuf[r, pl.ds(q * 16, 16)] * rec
                    dbuf[r, pl.ds(q * 16, 16)] = jnp.maximum(v, 0.0)
            return cc
        lax.fori_loop(0, RC // 16, grp, 0)
        pltpu.sync_copy(dbuf, out_ref.at[pl.ds(cid * NP + r0, RC)])
        return c
    lax.fori_loop(0, VT // RC, div_v, 0)


_sc_agg = functools.partial(
    pl.kernel,
    out_type=jax.ShapeDtypeStruct((NC * NP, DH), jnp.float32),
    mesh=plsc.VectorSubcoreMesh(core_axis_name="c", subcore_axis_name="s"),
    compiler_params=pltpu.CompilerParams(use_tc_tiling_on_sc=False),
    scratch_types=[
        pltpu.VMEM_SHARED((MP, DH), jnp.float32),
        pltpu.VMEM_SHARED((NP, DH), jnp.float32),
        pltpu.VMEM_SHARED((MP,), jnp.float32),
        pltpu.VMEM_SHARED((NP,), jnp.float32),
        pltpu.VMEM((B,), jnp.int32),
        pltpu.VMEM((B,), jnp.int32),
        pltpu.VMEM((B,), jnp.int32),
        pltpu.VMEM((B, DH), jnp.float32),
        pltpu.VMEM((B,), jnp.int32),
        pltpu.VMEM((B,), jnp.int32),
        pltpu.VMEM((B,), jnp.int32),
        pltpu.VMEM((B, DH), jnp.float32),
        pltpu.VMEM((B,), jnp.float32),
        pltpu.VMEM((RC, DH), jnp.float32),
        pltpu.VMEM((RC, DH), jnp.float32),
        pltpu.VMEM((RC,), jnp.float32),
        pltpu.VMEM((VT,), jnp.float32),
        pltpu.SemaphoreType.DMA,
        pltpu.SemaphoreType.DMA,
    ],
)(_sc_body)


def kernel(X, v_idx, e_idx, W, b):
    vi = v_idx.astype(jnp.int32)
    ei = e_idx.astype(jnp.int32)
    H2 = _project(X, W, b)
    out2 = _sc_agg(H2, vi, ei)
    return jnp.concatenate([out2[:N], out2[NP:NP + N]], axis=1)


# speedup vs baseline: 1.0135x; 1.0135x over previous
import functools

import jax
import jax.numpy as jnp
from jax import lax
from jax.experimental import pallas as pl
from jax.experimental.pallas import tpu as pltpu
from jax.experimental.pallas import tpu_sc as plsc

N = 10000
M = 5000
NNZ = 320000
DIN = 128
DOUT = 128
DH = DOUT // 2
NC = 2
NS = 16
MP = 5120
NP = 10240
PT = NNZ // NS
B = 400
NK = PT // B
RC = 64
ET = MP // NS
VT = NP // NS


def _mm_body(x_ref, w_ref, b_ref, o_ref):
    o_ref[...] = lax.dot_general(
        x_ref[...], w_ref[...], (((1,), (1,)), ((), ())),
        preferred_element_type=jnp.float32) + b_ref[0]


def _project(X, W, b):
    blk = 1000
    nblk = N // blk
    return pl.pallas_call(
        _mm_body,
        grid=(NC, nblk),
        in_specs=[
            pl.BlockSpec((blk, DIN), lambda c, i: (i, 0)),
            pl.BlockSpec((DH, DIN), lambda c, i: (c, 0)),
            pl.BlockSpec((1, 1, DH), lambda c, i: (c, 0, 0)),
        ],
        out_specs=pl.BlockSpec((blk, DH), lambda c, i: (c * nblk + i, 0)),
        out_shape=jax.ShapeDtypeStruct((NC * N, DH), jnp.float32),
    )(X, W, b.reshape(NC, 1, DH))


def _sc_body(h_ref, vi_ref, ei_ref, out_ref,
             e_acc, v_acc, e_cnt, v_cnt,
             vidx0, vadj0, eidx0, rows0, vidx1, vadj1, eidx1, rows1,
             ones, zbuf, dbuf, cbuf, z1,
             sem_g, sem_s0, sem_c):
    cid = lax.axis_index("c")
    sid = lax.axis_index("s")

    one16 = jnp.full((16,), 1.0, jnp.float32)
    zero16 = jnp.zeros((16,), jnp.float32)

    def init_ones(i, c):
        ones[pl.ds(i * 16, 16)] = one16
        return c
    lax.fori_loop(0, B // 16, init_ones, 0)

    def init_z(i, c):
        zbuf[i // 4, pl.ds((i % 4) * 16, 16)] = zero16
        return c
    lax.fori_loop(0, RC * 4, init_z, 0)

    def init_z1(i, c):
        z1[pl.ds(i * 16, 16)] = zero16
        return c
    lax.fori_loop(0, VT // 16, init_z1, 0)

    def zero_e(j, c):
        pltpu.sync_copy(zbuf, e_acc.at[pl.ds(sid * ET + j * RC, RC)])
        return c
    lax.fori_loop(0, ET // RC, zero_e, 0)

    def zero_v(j, c):
        pltpu.sync_copy(zbuf, v_acc.at[pl.ds(sid * VT + j * RC, RC)])
        return c
    lax.fori_loop(0, VT // RC, zero_v, 0)

    pltpu.sync_copy(z1.at[pl.ds(0, ET)], e_cnt.at[pl.ds(sid * ET, ET)])
    pltpu.sync_copy(z1, v_cnt.at[pl.ds(sid * VT, VT)])
    plsc.subcore_barrier()

    pair0 = sid * PT
    off = cid * N

    def load_adj(k, vidx, vadj, eidx):
        base = pair0 + k * B
        pltpu.sync_copy(vi_ref.at[pl.ds(base, B)], vidx)
        pltpu.sync_copy(ei_ref.at[pl.ds(base, B)], eidx)

        def adj(i, cc):
            vadj[pl.ds(i * 16, 16)] = vidx[pl.ds(i * 16, 16)] + off
            return cc
        lax.fori_loop(0, B // 16, adj, 0)

    def chunk_a(k, vidx, vadj, eidx, rows, sem_gc,
                nvidx, nvadj, neidx, nrows, nsem_g):
        @pl.when(k + 1 < NK)
        def _():
            load_adj(k + 1, nvidx, nvadj, neidx)
            pltpu.async_copy(h_ref.at[nvadj], nrows, nsem_g)
        pltpu.async_copy(ones, v_cnt.at[vidx], sem_c, add=True)
        pltpu.async_copy(ones, e_cnt.at[eidx], sem_c, add=True)
        pltpu.make_async_copy(h_ref.at[vadj], rows, sem_gc).wait()
        pltpu.sync_copy(rows, e_acc.at[eidx], add=True)
        pltpu.make_async_copy(ones, v_cnt.at[vidx], sem_c).wait()
        pltpu.make_async_copy(ones, e_cnt.at[eidx], sem_c).wait()

    load_adj(0, vidx0, vadj0, eidx0)
    pltpu.async_copy(h_ref.at[vadj0], rows0, sem_g)

    def pair_a(i, c):
        chunk_a(2 * i, vidx0, vadj0, eidx0, rows0, sem_g,
                vidx1, vadj1, eidx1, rows1, sem_s0)
        chunk_a(2 * i + 1, vidx1, vadj1, eidx1, rows1, sem_s0,
                vidx0, vadj0, eidx0, rows0, sem_g)
        return c
    lax.fori_loop(0, NK // 2, pair_a, 0)
    plsc.subcore_barrier()

    def div_e(j, c):
        r0 = sid * ET + j * RC
        pltpu.sync_copy(e_acc.at[pl.ds(r0, RC)], dbuf)
        pltpu.sync_copy(e_cnt.at[pl.ds(r0, RC)], cbuf)

        def grp(g, cc):
            recv = 1.0 / jnp.maximum(cbuf[pl.ds(g * 16, 16)], 1.0)
            for r2 in range(16):
                r = g * 16 + r2
                rec = recv[r2]
                for q in range(DH // 16):
                    dbuf[r, pl.ds(q * 16, 16)] = dbuf[r, pl.ds(q * 16, 16)] * rec
            return cc
        lax.fori_loop(0, RC // 16, grp, 0)
        pltpu.sync_copy(dbuf, e_acc.at[pl.ds(r0, RC)])
        return c
    lax.fori_loop(0, ET // RC, div_e, 0)
    plsc.subcore_barrier()

    def load_b(k, vidx, eidx):
        base = pair0 + k * B
        pltpu.sync_copy(vi_ref.at[pl.ds(base, B)], vidx)
        pltpu.sync_copy(ei_ref.at[pl.ds(base, B)], eidx)

    def chunk_b(k, vidx, eidx, rows, sem_gc, nvidx, neidx, nrows, nsem_g):
        @pl.when(k + 1 < NK)
        def _():
            load_b(k + 1, nvidx, neidx)
            pltpu.async_copy(e_acc.at[neidx], nrows, nsem_g)
        pltpu.make_async_copy(e_acc.at[eidx], rows, sem_gc).wait()
        pltpu.sync_copy(rows, v_acc.at[vidx], add=True)

    load_b(0, vidx0, eidx0)
    pltpu.async_copy(e_acc.at[eidx0], rows0, sem_g)

    def pair_b(i, c):
        chunk_b(2 * i, vidx0, eidx0, rows0, sem_g,
                vidx1, eidx1, rows1, sem_s0)
        chunk_b(2 * i + 1, vidx1, eidx1, rows1, sem_s0,
                vidx0, eidx0, rows0, sem_g)
        return c
    lax.fori_loop(0, NK // 2, pair_b, 0)
    plsc.subcore_barrier()

    def div_v(j, c):
        r0 = sid * VT + j * RC
        pltpu.sync_copy(v_acc.at[pl.ds(r0, RC)], dbuf)
        pltpu.sync_copy(v_cnt.at[pl.ds(r0, RC)], cbuf)

        def grp(g, cc):
            recv = 1.0 / jnp.maximum(cbuf[pl.ds(g * 16, 16)], 1.0)
            for r2 in range(16):
                r = g * 16 + r2
                rec = recv[r2]
                for q in range(DH // 16):
                    v = dbuf[r, pl.ds(q * 16, 16)] * rec
                    dbuf[r, pl.ds(q * 16, 16)] = jnp.maximum(v, 0.0)
            return cc
        lax.fori_loop(0, RC // 16, grp, 0)
        pltpu.sync_copy(dbuf, out_ref.at[pl.ds(cid * NP + r0, RC)])
        return c
    lax.fori_loop(0, VT // RC, div_v, 0)


_sc_agg = functools.partial(
    pl.kernel,
    out_type=jax.ShapeDtypeStruct((NC * NP, DH), jnp.float32),
    mesh=plsc.VectorSubcoreMesh(core_axis_name="c", subcore_axis_name="s"),
    compiler_params=pltpu.CompilerParams(use_tc_tiling_on_sc=False),
    scratch_types=[
        pltpu.VMEM_SHARED((MP, DH), jnp.float32),
        pltpu.VMEM_SHARED((NP, DH), jnp.float32),
        pltpu.VMEM_SHARED((MP,), jnp.float32),
        pltpu.VMEM_SHARED((NP,), jnp.float32),
        pltpu.VMEM((B,), jnp.int32),
        pltpu.VMEM((B,), jnp.int32),
        pltpu.VMEM((B,), jnp.int32),
        pltpu.VMEM((B, DH), jnp.float32),
        pltpu.VMEM((B,), jnp.int32),
        pltpu.VMEM((B,), jnp.int32),
        pltpu.VMEM((B,), jnp.int32),
        pltpu.VMEM((B, DH), jnp.float32),
        pltpu.VMEM((B,), jnp.float32),
        pltpu.VMEM((RC, DH), jnp.float32),
        pltpu.VMEM((RC, DH), jnp.float32),
        pltpu.VMEM((RC,), jnp.float32),
        pltpu.VMEM((VT,), jnp.float32),
        pltpu.SemaphoreType.DMA,
        pltpu.SemaphoreType.DMA,
        pltpu.SemaphoreType.DMA,
    ],
)(_sc_body)


def kernel(X, v_idx, e_idx, W, b):
    vi = v_idx.astype(jnp.int32)
    ei = e_idx.astype(jnp.int32)
    H2 = _project(X, W, b)
    out2 = _sc_agg(H2, vi, ei)
    return jnp.concatenate([out2[:N], out2[NP:NP + N]], axis=1)


# speedup vs baseline: 1.1213x; 1.1064x over previous
import functools

import jax
import jax.numpy as jnp
from jax import lax
from jax.experimental import pallas as pl
from jax.experimental.pallas import tpu as pltpu
from jax.experimental.pallas import tpu_sc as plsc

N = 10000
M = 5000
NNZ = 320000
DIN = 128
DOUT = 128
DH = DOUT // 2
NC = 2
NS = 16
MP = 5120
NP = 10240
PT = NNZ // NS
B = 400
NK = PT // B
RC = 64
ET = MP // NS
VT = NP // NS


def _mm_body(x_ref, w_ref, b_ref, o_ref):
    o_ref[...] = lax.dot_general(
        x_ref[...], w_ref[...], (((1,), (1,)), ((), ())),
        preferred_element_type=jnp.float32) + b_ref[0]


def _project(X, W, b):
    blk = 1000
    nblk = N // blk
    return pl.pallas_call(
        _mm_body,
        grid=(NC, nblk),
        in_specs=[
            pl.BlockSpec((blk, DIN), lambda c, i: (i, 0)),
            pl.BlockSpec((DH, DIN), lambda c, i: (c, 0)),
            pl.BlockSpec((1, 1, DH), lambda c, i: (c, 0, 0)),
        ],
        out_specs=pl.BlockSpec((blk, DH), lambda c, i: (c * nblk + i, 0)),
        out_shape=jax.ShapeDtypeStruct((NC * N, DH), jnp.float32),
    )(X, W, b.reshape(NC, 1, DH))


def _sc_body(h_ref, vi_ref, ei_ref, out_ref,
             e_acc, v_acc, e_cnt, v_cnt,
             vidx0, vadj0, eidx0, rows0, vidx1, vadj1, eidx1, rows1,
             ones, zbuf, dbuf, cbuf, z1,
             sem_g, sem_s0, sem_c, sem_i):
    cid = lax.axis_index("c")
    sid = lax.axis_index("s")

    one16 = jnp.full((16,), 1.0, jnp.float32)
    zero16 = jnp.zeros((16,), jnp.float32)

    def init_ones(i, c):
        ones[pl.ds(i * 16, 16)] = one16
        return c
    lax.fori_loop(0, B // 16, init_ones, 0)

    def init_z(i, c):
        zbuf[i // 4, pl.ds((i % 4) * 16, 16)] = zero16
        return c
    lax.fori_loop(0, RC * 4, init_z, 0)

    def init_z1(i, c):
        z1[pl.ds(i * 16, 16)] = zero16
        return c
    lax.fori_loop(0, VT // 16, init_z1, 0)

    def zero_e(j, c):
        pltpu.sync_copy(zbuf, e_acc.at[pl.ds(sid * ET + j * RC, RC)])
        return c
    lax.fori_loop(0, ET // RC, zero_e, 0)

    def zero_v(j, c):
        pltpu.sync_copy(zbuf, v_acc.at[pl.ds(sid * VT + j * RC, RC)])
        return c
    lax.fori_loop(0, VT // RC, zero_v, 0)

    pltpu.sync_copy(z1.at[pl.ds(0, ET)], e_cnt.at[pl.ds(sid * ET, ET)])
    pltpu.sync_copy(z1, v_cnt.at[pl.ds(sid * VT, VT)])
    plsc.subcore_barrier()

    pair0 = sid * PT
    off = cid * N

    def load_adj(k, vidx, vadj, eidx):
        base = pair0 + k * B
        ca = pltpu.async_copy(vi_ref.at[pl.ds(base, B)], vidx, sem_i)
        cb = pltpu.async_copy(ei_ref.at[pl.ds(base, B)], eidx, sem_i)
        ca.wait()
        cb.wait()

        def adj(i, cc):
            vadj[pl.ds(i * 16, 16)] = vidx[pl.ds(i * 16, 16)] + off
            return cc
        lax.fori_loop(0, B // 16, adj, 0)

    def chunk_a(k, vidx, vadj, eidx, rows, sem_gc,
                nvidx, nvadj, neidx, nrows, nsem_g):
        @pl.when(k + 1 < NK)
        def _():
            load_adj(k + 1, nvidx, nvadj, neidx)
            pltpu.async_copy(h_ref.at[nvadj], nrows, nsem_g)
        pltpu.async_copy(ones, v_cnt.at[vidx], sem_c, add=True)
        pltpu.async_copy(ones, e_cnt.at[eidx], sem_c, add=True)
        pltpu.make_async_copy(h_ref.at[vadj], rows, sem_gc).wait()
        pltpu.sync_copy(rows, e_acc.at[eidx], add=True)
        pltpu.make_async_copy(ones, v_cnt.at[vidx], sem_c).wait()
        pltpu.make_async_copy(ones, e_cnt.at[eidx], sem_c).wait()

    load_adj(0, vidx0, vadj0, eidx0)
    pltpu.async_copy(h_ref.at[vadj0], rows0, sem_g)

    def pair_a(i, c):
        chunk_a(2 * i, vidx0, vadj0, eidx0, rows0, sem_g,
                vidx1, vadj1, eidx1, rows1, sem_s0)
        chunk_a(2 * i + 1, vidx1, vadj1, eidx1, rows1, sem_s0,
                vidx0, vadj0, eidx0, rows0, sem_g)
        return c
    lax.fori_loop(0, NK // 2, pair_a, 0)
    plsc.subcore_barrier()

    def div_e(j, c):
        r0 = sid * ET + j * RC
        pltpu.sync_copy(e_acc.at[pl.ds(r0, RC)], dbuf)
        pltpu.sync_copy(e_cnt.at[pl.ds(r0, RC)], cbuf)

        def grp(g, cc):
            recv = 1.0 / jnp.maximum(cbuf[pl.ds(g * 16, 16)], 1.0)
            for r2 in range(16):
                r = g * 16 + r2
                rec = recv[r2]
                for q in range(DH // 16):
                    dbuf[r, pl.ds(q * 16, 16)] = dbuf[r, pl.ds(q * 16, 16)] * rec
            return cc
        lax.fori_loop(0, RC // 16, grp, 0)
        pltpu.sync_copy(dbuf, e_acc.at[pl.ds(r0, RC)])
        return c
    lax.fori_loop(0, ET // RC, div_e, 0)
    plsc.subcore_barrier()

    def load_b(k, vidx, eidx):
        base = pair0 + k * B
        ca = pltpu.async_copy(vi_ref.at[pl.ds(base, B)], vidx, sem_i)
        cb = pltpu.async_copy(ei_ref.at[pl.ds(base, B)], eidx, sem_i)
        ca.wait()
        cb.wait()

    def chunk_b(k, vidx, eidx, rows, sem_gc, nvidx, neidx, nrows, nsem_g):
        @pl.when(k + 1 < NK)
        def _():
            load_b(k + 1, nvidx, neidx)
            pltpu.async_copy(e_acc.at[neidx], nrows, nsem_g)
        pltpu.make_async_copy(e_acc.at[eidx], rows, sem_gc).wait()
        pltpu.sync_copy(rows, v_acc.at[vidx], add=True)

    load_b(0, vidx0, eidx0)
    pltpu.async_copy(e_acc.at[eidx0], rows0, sem_g)

    def pair_b(i, c):
        chunk_b(2 * i, vidx0, eidx0, rows0, sem_g,
                vidx1, eidx1, rows1, sem_s0)
        chunk_b(2 * i + 1, vidx1, eidx1, rows1, sem_s0,
                vidx0, eidx0, rows0, sem_g)
        return c
    lax.fori_loop(0, NK // 2, pair_b, 0)
    plsc.subcore_barrier()

    def div_v(j, c):
        r0 = sid * VT + j * RC
        pltpu.sync_copy(v_acc.at[pl.ds(r0, RC)], dbuf)
        pltpu.sync_copy(v_cnt.at[pl.ds(r0, RC)], cbuf)

        def grp(g, cc):
            recv = 1.0 / jnp.maximum(cbuf[pl.ds(g * 16, 16)], 1.0)
            for r2 in range(16):
                r = g * 16 + r2
                rec = recv[r2]
                for q in range(DH // 16):
                    v = dbuf[r, pl.ds(q * 16, 16)] * rec
                    dbuf[r, pl.ds(q * 16, 16)] = jnp.maximum(v, 0.0)
            return cc
        lax.fori_loop(0, RC // 16, grp, 0)
        pltpu.sync_copy(dbuf, out_ref.at[pl.ds(cid * NP + r0, RC)])
        return c
    lax.fori_loop(0, VT // RC, div_v, 0)


_sc_agg = functools.partial(
    pl.kernel,
    out_type=jax.ShapeDtypeStruct((NC * NP, DH), jnp.float32),
    mesh=plsc.VectorSubcoreMesh(core_axis_name="c", subcore_axis_name="s"),
    compiler_params=pltpu.CompilerParams(use_tc_tiling_on_sc=False),
    scratch_types=[
        pltpu.VMEM_SHARED((MP, DH), jnp.float32),
        pltpu.VMEM_SHARED((NP, DH), jnp.float32),
        pltpu.VMEM_SHARED((MP,), jnp.float32),
        pltpu.VMEM_SHARED((NP,), jnp.float32),
        pltpu.VMEM((B,), jnp.int32),
        pltpu.VMEM((B,), jnp.int32),
        pltpu.VMEM((B,), jnp.int32),
        pltpu.VMEM((B, DH), jnp.float32),
        pltpu.VMEM((B,), jnp.int32),
        pltpu.VMEM((B,), jnp.int32),
        pltpu.VMEM((B,), jnp.int32),
        pltpu.VMEM((B, DH), jnp.float32),
        pltpu.VMEM((B,), jnp.float32),
        pltpu.VMEM((RC, DH), jnp.float32),
        pltpu.VMEM((RC, DH), jnp.float32),
        pltpu.VMEM((RC,), jnp.float32),
        pltpu.VMEM((VT,), jnp.float32),
        pltpu.SemaphoreType.DMA,
        pltpu.SemaphoreType.DMA,
        pltpu.SemaphoreType.DMA,
        pltpu.SemaphoreType.DMA,
    ],
)(_sc_body)


def kernel(X, v_idx, e_idx, W, b):
    vi = v_idx.astype(jnp.int32)
    ei = e_idx.astype(jnp.int32)
    H2 = _project(X, W, b)
    out2 = _sc_agg(H2, vi, ei)
    return jnp.concatenate([out2[:N], out2[NP:NP + N]], axis=1)


# speedup vs baseline: 1.1236x; 1.0020x over previous
import functools

import jax
import jax.numpy as jnp
from jax import lax
from jax.experimental import pallas as pl
from jax.experimental.pallas import tpu as pltpu
from jax.experimental.pallas import tpu_sc as plsc

N = 10000
M = 5000
NNZ = 320000
DIN = 128
DOUT = 128
DH = DOUT // 2
NC = 2
NS = 16
MP = 5120
NP = 10240
PT = NNZ // NS
B = 400
NK = PT // B
RC = 64
ET = MP // NS
VT = NP // NS


def _mm_body(x_ref, w_ref, b_ref, o_ref):
    o_ref[...] = lax.dot_general(
        x_ref[...], w_ref[...], (((1,), (1,)), ((), ())),
        preferred_element_type=jnp.float32) + b_ref[0]


def _project(X, W, b):
    blk = 1000
    nblk = N // blk
    return pl.pallas_call(
        _mm_body,
        grid=(NC, nblk),
        in_specs=[
            pl.BlockSpec((blk, DIN), lambda c, i: (i, 0)),
            pl.BlockSpec((DH, DIN), lambda c, i: (c, 0)),
            pl.BlockSpec((1, 1, DH), lambda c, i: (c, 0, 0)),
        ],
        out_specs=pl.BlockSpec((blk, DH), lambda c, i: (c * nblk + i, 0)),
        out_shape=jax.ShapeDtypeStruct((NC * N, DH), jnp.float32),
    )(X, W, b.reshape(NC, 1, DH))


def _sc_body(h_ref, vi_ref, ei_ref, out_ref,
             e_acc, v_acc, e_cnt, v_cnt,
             vidx0, vadj0, eidx0, rows0, vidx1, vadj1, eidx1, rows1,
             ones, zbuf, dbuf, cbuf, z1,
             sem_g, sem_s0, sem_c, sem_i, sem_i2):
    cid = lax.axis_index("c")
    sid = lax.axis_index("s")

    one16 = jnp.full((16,), 1.0, jnp.float32)
    zero16 = jnp.zeros((16,), jnp.float32)

    def init_ones(i, c):
        ones[pl.ds(i * 16, 16)] = one16
        return c
    lax.fori_loop(0, B // 16, init_ones, 0)

    def init_z(i, c):
        zbuf[i // 4, pl.ds((i % 4) * 16, 16)] = zero16
        return c
    lax.fori_loop(0, RC * 4, init_z, 0)

    def init_z1(i, c):
        z1[pl.ds(i * 16, 16)] = zero16
        return c
    lax.fori_loop(0, VT // 16, init_z1, 0)

    def zero_e(j, c):
        pltpu.sync_copy(zbuf, e_acc.at[pl.ds(sid * ET + j * RC, RC)])
        return c
    lax.fori_loop(0, ET // RC, zero_e, 0)

    def zero_v(j, c):
        pltpu.sync_copy(zbuf, v_acc.at[pl.ds(sid * VT + j * RC, RC)])
        return c
    lax.fori_loop(0, VT // RC, zero_v, 0)

    pltpu.sync_copy(z1.at[pl.ds(0, ET)], e_cnt.at[pl.ds(sid * ET, ET)])
    pltpu.sync_copy(z1, v_cnt.at[pl.ds(sid * VT, VT)])
    plsc.subcore_barrier()

    pair0 = sid * PT
    off = cid * N

    def load_adj(k, vidx, vadj, eidx):
        base = pair0 + k * B
        ca = pltpu.async_copy(vi_ref.at[pl.ds(base, B)], vidx, sem_i)
        cb = pltpu.async_copy(ei_ref.at[pl.ds(base, B)], eidx, sem_i2)
        ca.wait()
        cb.wait()

        def adj(i, cc):
            vadj[pl.ds(i * 16, 16)] = vidx[pl.ds(i * 16, 16)] + off
            return cc
        lax.fori_loop(0, B // 16, adj, 0)

    def chunk_a(k, vidx, vadj, eidx, rows, sem_gc,
                nvidx, nvadj, neidx, nrows, nsem_g):
        @pl.when(k + 1 < NK)
        def _():
            load_adj(k + 1, nvidx, nvadj, neidx)
            pltpu.async_copy(h_ref.at[nvadj], nrows, nsem_g)
        pltpu.async_copy(ones, v_cnt.at[vidx], sem_c, add=True)
        pltpu.async_copy(ones, e_cnt.at[eidx], sem_c, add=True)
        pltpu.make_async_copy(h_ref.at[vadj], rows, sem_gc).wait()
        pltpu.sync_copy(rows, e_acc.at[eidx], add=True)
        pltpu.make_async_copy(ones, v_cnt.at[vidx], sem_c).wait()
        pltpu.make_async_copy(ones, e_cnt.at[eidx], sem_c).wait()

    load_adj(0, vidx0, vadj0, eidx0)
    pltpu.async_copy(h_ref.at[vadj0], rows0, sem_g)

    def pair_a(i, c):
        chunk_a(2 * i, vidx0, vadj0, eidx0, rows0, sem_g,
                vidx1, vadj1, eidx1, rows1, sem_s0)
        chunk_a(2 * i + 1, vidx1, vadj1, eidx1, rows1, sem_s0,
                vidx0, vadj0, eidx0, rows0, sem_g)
        return c
    lax.fori_loop(0, NK // 2, pair_a, 0)
    plsc.subcore_barrier()

    def div_e(j, c):
        r0 = sid * ET + j * RC
        pltpu.sync_copy(e_acc.at[pl.ds(r0, RC)], dbuf)
        pltpu.sync_copy(e_cnt.at[pl.ds(r0, RC)], cbuf)

        def grp(g, cc):
            recv = 1.0 / jnp.maximum(cbuf[pl.ds(g * 16, 16)], 1.0)
            for r2 in range(16):
                r = g * 16 + r2
                rec = recv[r2]
                for q in range(DH // 16):
                    dbuf[r, pl.ds(q * 16, 16)] = dbuf[r, pl.ds(q * 16, 16)] * rec
            return cc
        lax.fori_loop(0, RC // 16, grp, 0)
        pltpu.sync_copy(dbuf, e_acc.at[pl.ds(r0, RC)])
        return c
    lax.fori_loop(0, ET // RC, div_e, 0)
    plsc.subcore_barrier()

    def load_b(k, vidx, eidx):
        base = pair0 + k * B
        ca = pltpu.async_copy(vi_ref.at[pl.ds(base, B)], vidx, sem_i)
        cb = pltpu.async_copy(ei_ref.at[pl.ds(base, B)], eidx, sem_i2)
        ca.wait()
        cb.wait()

    def chunk_b(k, vidx, eidx, rows, sem_gc, nvidx, neidx, nrows, nsem_g):
        @pl.when(k + 1 < NK)
        def _():
            load_b(k + 1, nvidx, neidx)
            pltpu.async_copy(e_acc.at[neidx], nrows, nsem_g)
        pltpu.make_async_copy(e_acc.at[eidx], rows, sem_gc).wait()
        pltpu.sync_copy(rows, v_acc.at[vidx], add=True)

    load_b(0, vidx0, eidx0)
    pltpu.async_copy(e_acc.at[eidx0], rows0, sem_g)

    def pair_b(i, c):
        chunk_b(2 * i, vidx0, eidx0, rows0, sem_g,
                vidx1, eidx1, rows1, sem_s0)
        chunk_b(2 * i + 1, vidx1, eidx1, rows1, sem_s0,
                vidx0, eidx0, rows0, sem_g)
        return c
    lax.fori_loop(0, NK // 2, pair_b, 0)
    plsc.subcore_barrier()

    def div_v(j, c):
        r0 = sid * VT + j * RC
        pltpu.sync_copy(v_acc.at[pl.ds(r0, RC)], dbuf)
        pltpu.sync_copy(v_cnt.at[pl.ds(r0, RC)], cbuf)

        def grp(g, cc):
            recv = 1.0 / jnp.maximum(cbuf[pl.ds(g * 16, 16)], 1.0)
            for r2 in range(16):
                r = g * 16 + r2
                rec = recv[r2]
                for q in range(DH // 16):
                    v = dbuf[r, pl.ds(q * 16, 16)] * rec
                    dbuf[r, pl.ds(q * 16, 16)] = jnp.maximum(v, 0.0)
            return cc
        lax.fori_loop(0, RC // 16, grp, 0)
        pltpu.sync_copy(dbuf, out_ref.at[pl.ds(cid * NP + r0, RC)])
        return c
    lax.fori_loop(0, VT // RC, div_v, 0)


_sc_agg = functools.partial(
    pl.kernel,
    out_type=jax.ShapeDtypeStruct((NC * NP, DH), jnp.float32),
    mesh=plsc.VectorSubcoreMesh(core_axis_name="c", subcore_axis_name="s"),
    compiler_params=pltpu.CompilerParams(use_tc_tiling_on_sc=False),
    scratch_types=[
        pltpu.VMEM_SHARED((MP, DH), jnp.float32),
        pltpu.VMEM_SHARED((NP, DH), jnp.float32),
        pltpu.VMEM_SHARED((MP,), jnp.float32),
        pltpu.VMEM_SHARED((NP,), jnp.float32),
        pltpu.VMEM((B,), jnp.int32),
        pltpu.VMEM((B,), jnp.int32),
        pltpu.VMEM((B,), jnp.int32),
        pltpu.VMEM((B, DH), jnp.float32),
        pltpu.VMEM((B,), jnp.int32),
        pltpu.VMEM((B,), jnp.int32),
        pltpu.VMEM((B,), jnp.int32),
        pltpu.VMEM((B, DH), jnp.float32),
        pltpu.VMEM((B,), jnp.float32),
        pltpu.VMEM((RC, DH), jnp.float32),
        pltpu.VMEM((RC, DH), jnp.float32),
        pltpu.VMEM((RC,), jnp.float32),
        pltpu.VMEM((VT,), jnp.float32),
        pltpu.SemaphoreType.DMA,
        pltpu.SemaphoreType.DMA,
        pltpu.SemaphoreType.DMA,
        pltpu.SemaphoreType.DMA,
        pltpu.SemaphoreType.DMA,
    ],
)(_sc_body)


def kernel(X, v_idx, e_idx, W, b):
    vi = v_idx.astype(jnp.int32)
    ei = e_idx.astype(jnp.int32)
    H2 = _project(X, W, b)
    out2 = _sc_agg(H2, vi, ei)
    return jnp.concatenate([out2[:N], out2[NP:NP + N]], axis=1)
